# Initial kernel scaffold; baseline (speedup 1.0000x reference)
import jax
import jax.numpy as jnp
from jax.experimental import pallas as pl


def kernel(x, keys, values, Wd, bd, Wu, Wp):
    raise NotImplementedError("write your pallas kernel here")


# speedup vs baseline: 7.6734x; 7.6734x over previous
import functools

import jax
import jax.numpy as jnp
import numpy as np
from jax import lax
from jax.experimental import pallas as pl
from jax.experimental.pallas import tpu as pltpu
from jax.experimental.pallas import tpu_sc as plsc

D_MODEL = 1024
MEM_N_KEYS = 512
HEADS = 8
KNN = 16
KEY_DIM = 288
VALUE_DIM = 128
Q_RANK = 512
NUM_BUCKETS = 18
BUCKET_DIM = 16
HM = HEADS * NUM_BUCKETS

TOK_BLK = 256

_PERM = np.zeros((HM, HM), np.float32)
for _t in range(NUM_BUCKETS):
    for _h in range(HEADS):
        _PERM[_t * HEADS + _h, _h * NUM_BUCKETS + _t] = 1.0


def _scores_topk_body(xT_ref, wd_ref, wu_ref, kk_ref, bd_ref, idx_ref, wts_ref):
    xT = xT_ref[...]
    h1 = jnp.dot(wd_ref[...], xT, preferred_element_type=jnp.float32)
    h1 = h1 + bd_ref[...]
    q = jnp.dot(wu_ref[...], h1, preferred_element_type=jnp.float32)
    s01 = jnp.dot(kk_ref[...], q, preferred_element_type=jnp.float32)

    deltas = []
    code = jnp.zeros((HEADS, TOK_BLK), jnp.int32)
    for t in range(NUM_BUCKETS):
        s0 = s01[t * HEADS:(t + 1) * HEADS, :]
        s1 = s01[HM + t * HEADS:HM + (t + 1) * HEADS, :]
        deltas.append(jnp.abs(s0 - s1))
        code = code | jnp.where(s1 > s0, jnp.int32(1 << t), jnp.int32(0))

    inf = jnp.full((HEADS, TOK_BLK), jnp.inf, jnp.float32)
    zero_i = jnp.zeros((HEADS, TOK_BLK), jnp.int32)
    pen = [jnp.zeros((HEADS, TOK_BLK), jnp.float32)] + [inf] * (KNN - 1)
    msk = [zero_i] * KNN
    for t in range(NUM_BUCKETS):
        d = deltas[t]
        bit = jnp.int32(1 << t)
        lo, lm = [], []
        for i in range(KNN):
            a_p = pen[i]
            b_p = pen[KNN - 1 - i] + d
            c = a_p <= b_p
            lo.append(jnp.where(c, a_p, b_p))
            lm.append(jnp.where(c, msk[i], msk[KNN - 1 - i] ^ bit))
        for dist in (8, 4, 2, 1):
            nlo, nlm = list(lo), list(lm)
            for i in range(KNN):
                if i & dist:
                    continue
                j = i + dist
                c = lo[i] <= lo[j]
                nlo[i] = jnp.where(c, lo[i], lo[j])
                nlm[i] = jnp.where(c, lm[i], lm[j])
                nlo[j] = jnp.where(c, lo[j], lo[i])
                nlm[j] = jnp.where(c, lm[j], lm[i])
            lo, lm = nlo, nlm
        pen, msk = lo, lm

    es = [jnp.exp(pen[0] - pen[i]) for i in range(KNN)]
    z = es[0]
    for i in range(1, KNN):
        z = z + es[i]
    rz = 1.0 / z
    for i in range(KNN):
        idx_ref[i * HEADS:(i + 1) * HEADS, :] = code ^ msk[i]
        wts_ref[i * HEADS:(i + 1) * HEADS, :] = es[i] * rz


def _scores_topk(xT, Wd, Wu, KK, bd2):
    T = xT.shape[1]
    grid = (T // TOK_BLK,)
    return pl.pallas_call(
        _scores_topk_body,
        grid=grid,
        in_specs=[
            pl.BlockSpec((D_MODEL, TOK_BLK), lambda i: (0, i)),
            pl.BlockSpec((Q_RANK, D_MODEL), lambda i: (0, 0)),
            pl.BlockSpec((HEADS * KEY_DIM, Q_RANK), lambda i: (0, 0)),
            pl.BlockSpec((2 * HM, HEADS * KEY_DIM), lambda i: (0, 0)),
            pl.BlockSpec((Q_RANK, 1), lambda i: (0, 0)),
        ],
        out_specs=[
            pl.BlockSpec((HEADS * KNN, TOK_BLK), lambda i: (0, i)),
            pl.BlockSpec((HEADS * KNN, TOK_BLK), lambda i: (0, i)),
        ],
        out_shape=[
            jax.ShapeDtypeStruct((HEADS * KNN, T), jnp.int32),
            jax.ShapeDtypeStruct((HEADS * KNN, T), jnp.float32),
        ],
    )(xT, Wd, Wu, KK, bd2)


NW = 32
CT = 4
BAG = HEADS * KNN


def _bag_body(idx_hbm, w_hbm, table_hbm, out_hbm, idx_v, w_v, rows_v, y_v, sem):
    wid = lax.axis_index("s") * 2 + lax.axis_index("c")
    T = idx_hbm.shape[0]
    tw = T // NW
    tok0 = wid * tw

    def chunk(ci, _):
        tok = tok0 + ci * CT
        pltpu.sync_copy(idx_hbm.at[pl.ds(tok, CT)], idx_v)
        pltpu.sync_copy(w_hbm.at[pl.ds(tok, CT)], w_v)
        handles = [
            pltpu.async_copy(table_hbm.at[idx_v.at[c]], rows_v.at[c], sem)
            for c in range(CT)
        ]
        for h in handles:
            h.wait()
        for c in range(CT):
            def grp(g, accs):
                w16 = w_v[c, pl.ds(g * 16, 16)]
                for r in range(16):
                    wr = jnp.take(w16, jnp.full((16,), r, jnp.int32))
                    row = g * 16 + r
                    accs = tuple(
                        accs[j] + wr * rows_v[c, row, pl.ds(j * 16, 16)]
                        for j in range(VALUE_DIM // 16)
                    )
                return accs
            accs = lax.fori_loop(
                0, BAG // 16, grp,
                tuple(jnp.zeros((16,), jnp.float32)
                      for _ in range(VALUE_DIM // 16)))
            for j in range(VALUE_DIM // 16):
                y_v[c, pl.ds(j * 16, 16)] = accs[j]
        pltpu.sync_copy(y_v, out_hbm.at[pl.ds(tok, CT)])
        return 0

    lax.fori_loop(0, tw // CT, chunk, 0)


def _bag(idx_t, w_t, values):
    T = idx_t.shape[0]
    mesh = plsc.VectorSubcoreMesh(core_axis_name="c", subcore_axis_name="s")
    return pl.kernel(
        _bag_body,
        out_type=jax.ShapeDtypeStruct((T, VALUE_DIM), jnp.float32),
        mesh=mesh,
        scratch_types=[
            pltpu.VMEM((CT, BAG), jnp.int32),
            pltpu.VMEM((CT, BAG), jnp.float32),
            pltpu.VMEM((CT, BAG, VALUE_DIM), jnp.float32),
            pltpu.VMEM((CT, VALUE_DIM), jnp.float32),
            pltpu.SemaphoreType.DMA,
        ],
    )(idx_t, w_t, values)


def _proj_body(y_ref, wp_ref, out_ref):
    out_ref[...] = jnp.dot(y_ref[...], wp_ref[...],
                           preferred_element_type=jnp.float32)


def _proj(y, WpT):
    T = y.shape[0]
    blk = 512
    return pl.pallas_call(
        _proj_body,
        grid=(T // blk,),
        in_specs=[
            pl.BlockSpec((blk, VALUE_DIM), lambda i: (i, 0)),
            pl.BlockSpec((VALUE_DIM, D_MODEL), lambda i: (0, 0)),
        ],
        out_specs=pl.BlockSpec((blk, D_MODEL), lambda i: (i, 0)),
        out_shape=jax.ShapeDtypeStruct((T, D_MODEL), jnp.float32),
    )(y, WpT)


def kernel(x, keys, values, Wd, bd, Wu, Wp):
    bsz, seq_len, _ = x.shape
    T = bsz * seq_len
    xT = x.reshape(T, D_MODEL).T

    E = jnp.asarray(_PERM)
    keys_r = keys.transpose(2, 0, 1, 3).reshape(2, HM, BUCKET_DIM)
    KK = (E[None, :, :, None] * keys_r[:, None, :, :]).reshape(2 * HM, HM * BUCKET_DIM)
    bd2 = bd.reshape(Q_RANK, 1)

    idx, wts = _scores_topk(xT, Wd, Wu, KK, bd2)
    idx_t = idx.T.reshape(T, BAG)
    w_t = wts.T.reshape(T, BAG)

    y = _bag(idx_t, w_t, values)
    out = _proj(y, Wp.T)
    return out.reshape(bsz, seq_len, D_MODEL)


# speedup vs baseline: 11.4887x; 1.4972x over previous
import functools

import jax
import jax.numpy as jnp
import numpy as np
from jax import lax
from jax.experimental import pallas as pl
from jax.experimental.pallas import tpu as pltpu
from jax.experimental.pallas import tpu_sc as plsc

D_MODEL = 1024
MEM_N_KEYS = 512
HEADS = 8
KNN = 16
KEY_DIM = 288
VALUE_DIM = 128
Q_RANK = 512
NUM_BUCKETS = 18
BUCKET_DIM = 16
HM = HEADS * NUM_BUCKETS

TOK_BLK = 256

_PERM = np.zeros((HM, HM), np.float32)
for _t in range(NUM_BUCKETS):
    for _h in range(HEADS):
        _PERM[_t * HEADS + _h, _h * NUM_BUCKETS + _t] = 1.0


def _scores_topk_body(xT_ref, wd_ref, wu_ref, kk_ref, bd_ref, idx_ref, wts_ref):
    xT = xT_ref[...]
    h1 = jnp.dot(wd_ref[...], xT, preferred_element_type=jnp.float32)
    h1 = h1 + bd_ref[...]
    q = jnp.dot(wu_ref[...], h1, preferred_element_type=jnp.float32)
    s01 = jnp.dot(kk_ref[...], q, preferred_element_type=jnp.float32)

    deltas = []
    code = jnp.zeros((HEADS, TOK_BLK), jnp.int32)
    for t in range(NUM_BUCKETS):
        s0 = s01[t * HEADS:(t + 1) * HEADS, :]
        s1 = s01[HM + t * HEADS:HM + (t + 1) * HEADS, :]
        deltas.append(jnp.abs(s0 - s1))
        code = code | jnp.where(s1 > s0, jnp.int32(1 << t), jnp.int32(0))

    inf = jnp.full((HEADS, TOK_BLK), jnp.inf, jnp.float32)
    zero_i = jnp.zeros((HEADS, TOK_BLK), jnp.int32)
    pen = [jnp.zeros((HEADS, TOK_BLK), jnp.float32)] + [inf] * (KNN - 1)
    msk = [zero_i] * KNN
    for t in range(NUM_BUCKETS):
        d = deltas[t]
        bit = jnp.int32(1 << t)
        lo, lm = [], []
        for i in range(KNN):
            a_p = pen[i]
            b_p = pen[KNN - 1 - i] + d
            c = a_p <= b_p
            lo.append(jnp.where(c, a_p, b_p))
            lm.append(jnp.where(c, msk[i], msk[KNN - 1 - i] ^ bit))
        for dist in (8, 4, 2, 1):
            nlo, nlm = list(lo), list(lm)
            for i in range(KNN):
                if i & dist:
                    continue
                j = i + dist
                c = lo[i] <= lo[j]
                nlo[i] = jnp.where(c, lo[i], lo[j])
                nlm[i] = jnp.where(c, lm[i], lm[j])
                nlo[j] = jnp.where(c, lo[j], lo[i])
                nlm[j] = jnp.where(c, lm[j], lm[i])
            lo, lm = nlo, nlm
        pen, msk = lo, lm

    es = [jnp.exp(pen[0] - pen[i]) for i in range(KNN)]
    z = es[0]
    for i in range(1, KNN):
        z = z + es[i]
    rz = 1.0 / z
    for i in range(KNN):
        idx_ref[i * HEADS:(i + 1) * HEADS, :] = code ^ msk[i]
        wts_ref[i * HEADS:(i + 1) * HEADS, :] = es[i] * rz


def _scores_topk(xT, Wd, Wu, KK, bd2):
    T = xT.shape[1]
    grid = (T // TOK_BLK,)
    return pl.pallas_call(
        _scores_topk_body,
        grid=grid,
        in_specs=[
            pl.BlockSpec((D_MODEL, TOK_BLK), lambda i: (0, i)),
            pl.BlockSpec((Q_RANK, D_MODEL), lambda i: (0, 0)),
            pl.BlockSpec((HEADS * KEY_DIM, Q_RANK), lambda i: (0, 0)),
            pl.BlockSpec((2 * HM, HEADS * KEY_DIM), lambda i: (0, 0)),
            pl.BlockSpec((Q_RANK, 1), lambda i: (0, 0)),
        ],
        out_specs=[
            pl.BlockSpec((HEADS * KNN, TOK_BLK), lambda i: (0, i)),
            pl.BlockSpec((HEADS * KNN, TOK_BLK), lambda i: (0, i)),
        ],
        out_shape=[
            jax.ShapeDtypeStruct((HEADS * KNN, T), jnp.int32),
            jax.ShapeDtypeStruct((HEADS * KNN, T), jnp.float32),
        ],
    )(xT, Wd, Wu, KK, bd2)


NW = 32
TW = 128
NB = 4
BAG = HEADS * KNN


def _bag_body(idx_hbm, w_hbm, table_hbm, out_hbm, idx_v, w_v, rows_v, y_v,
              sem0, sem1, sem2, sem3):
    wid = lax.axis_index("s") * 2 + lax.axis_index("c")
    tok0 = wid * TW
    sems = (sem0, sem1, sem2, sem3)

    pltpu.sync_copy(idx_hbm.at[pl.ds(tok0, TW)], idx_v)
    pltpu.sync_copy(w_hbm.at[pl.ds(tok0, TW)], w_v)

    def fire(ci, b, sem):
        pltpu.async_copy(table_hbm.at[idx_v.at[ci]], rows_v.at[b], sem)

    for j in range(NB - 1):
        fire(j, j, sems[j])

    def quad(p, _):
        for b in range(NB):
            ci = p * NB + b

            @pl.when(ci + NB - 1 < TW)
            def _(ci=ci, b=b):
                fire(ci + NB - 1, (b + NB - 1) % NB, sems[(b + NB - 1) % NB])

            pltpu.make_async_copy(
                table_hbm.at[idx_v.at[ci]], rows_v.at[b], sems[b]).wait()

            def grp(g, accs, ci=ci, b=b):
                w16 = w_v[ci, pl.ds(g * 16, 16)]
                for r in range(16):
                    wr = jnp.take(w16, jnp.full((16,), r, jnp.int32))
                    row = g * 16 + r
                    accs = tuple(
                        accs[j] + wr * rows_v[b, row, pl.ds(j * 16, 16)]
                        for j in range(VALUE_DIM // 16)
                    )
                return accs

            accs = lax.fori_loop(
                0, BAG // 16, grp,
                tuple(jnp.zeros((16,), jnp.float32)
                      for _ in range(VALUE_DIM // 16)))
            for j in range(VALUE_DIM // 16):
                y_v[0, pl.ds(j * 16, 16)] = accs[j]
            pltpu.sync_copy(y_v, out_hbm.at[pl.ds(tok0 + ci, 1)])
        return 0

    lax.fori_loop(0, TW // NB, quad, 0)


def _bag(idx_t, w_t, values):
    T = idx_t.shape[0]
    mesh = plsc.VectorSubcoreMesh(core_axis_name="c", subcore_axis_name="s")
    return pl.kernel(
        _bag_body,
        out_type=jax.ShapeDtypeStruct((T, VALUE_DIM), jnp.float32),
        mesh=mesh,
        scratch_types=[
            pltpu.VMEM((TW, BAG), jnp.int32),
            pltpu.VMEM((TW, BAG), jnp.float32),
            pltpu.VMEM((NB, BAG, VALUE_DIM), jnp.float32),
            pltpu.VMEM((1, VALUE_DIM), jnp.float32),
            pltpu.SemaphoreType.DMA,
            pltpu.SemaphoreType.DMA,
            pltpu.SemaphoreType.DMA,
            pltpu.SemaphoreType.DMA,
        ],
    )(idx_t, w_t, values)


def _proj_body(y_ref, wp_ref, out_ref):
    out_ref[...] = jnp.dot(y_ref[...], wp_ref[...],
                           preferred_element_type=jnp.float32)


def _proj(y, WpT):
    T = y.shape[0]
    blk = 512
    return pl.pallas_call(
        _proj_body,
        grid=(T // blk,),
        in_specs=[
            pl.BlockSpec((blk, VALUE_DIM), lambda i: (i, 0)),
            pl.BlockSpec((VALUE_DIM, D_MODEL), lambda i: (0, 0)),
        ],
        out_specs=pl.BlockSpec((blk, D_MODEL), lambda i: (i, 0)),
        out_shape=jax.ShapeDtypeStruct((T, D_MODEL), jnp.float32),
    )(y, WpT)


def kernel(x, keys, values, Wd, bd, Wu, Wp):
    bsz, seq_len, _ = x.shape
    T = bsz * seq_len
    xT = x.reshape(T, D_MODEL).T

    E = jnp.asarray(_PERM)
    keys_r = keys.transpose(2, 0, 1, 3).reshape(2, HM, BUCKET_DIM)
    KK = (E[None, :, :, None] * keys_r[:, None, :, :]).reshape(2 * HM, HM * BUCKET_DIM)
    bd2 = bd.reshape(Q_RANK, 1)

    idx, wts = _scores_topk(xT, Wd, Wu, KK, bd2)
    idx_t = idx.T.reshape(T, BAG)
    w_t = wts.T.reshape(T, BAG)

    y = _bag(idx_t, w_t, values)
    out = _proj(y, Wp.T)
    return out.reshape(bsz, seq_len, D_MODEL)


# speedup vs baseline: 12.6333x; 1.0996x over previous
import functools

import jax
import jax.numpy as jnp
import numpy as np
from jax import lax
from jax.experimental import pallas as pl
from jax.experimental.pallas import tpu as pltpu
from jax.experimental.pallas import tpu_sc as plsc

D_MODEL = 1024
MEM_N_KEYS = 512
HEADS = 8
KNN = 16
KEY_DIM = 288
VALUE_DIM = 128
Q_RANK = 512
NUM_BUCKETS = 18
BUCKET_DIM = 16
HM = HEADS * NUM_BUCKETS

TOK_BLK = 256

_PERM = np.zeros((HM, HM), np.float32)
for _t in range(NUM_BUCKETS):
    for _h in range(HEADS):
        _PERM[_t * HEADS + _h, _h * NUM_BUCKETS + _t] = 1.0


def _scores_topk_body(x_ref, wd_ref, wu_ref, kk_ref, bd_ref, idx_ref, wts_ref):
    h1 = lax.dot_general(wd_ref[...], x_ref[...],
                         dimension_numbers=(((1,), (1,)), ((), ())),
                         preferred_element_type=jnp.float32)
    h1 = h1 + bd_ref[...]
    q = jnp.dot(wu_ref[...], h1, preferred_element_type=jnp.float32)
    s01 = jnp.dot(kk_ref[...], q, preferred_element_type=jnp.float32)

    deltas = []
    code = jnp.zeros((HEADS, TOK_BLK), jnp.int32)
    for t in range(NUM_BUCKETS):
        s0 = s01[t * HEADS:(t + 1) * HEADS, :]
        s1 = s01[HM + t * HEADS:HM + (t + 1) * HEADS, :]
        deltas.append(jnp.abs(s0 - s1))
        code = code | jnp.where(s1 > s0, jnp.int32(1 << t), jnp.int32(0))

    inf = jnp.full((HEADS, TOK_BLK), jnp.inf, jnp.float32)
    zero_i = jnp.zeros((HEADS, TOK_BLK), jnp.int32)
    pen = [jnp.zeros((HEADS, TOK_BLK), jnp.float32)] + [inf] * (KNN - 1)
    msk = [zero_i] * KNN
    for t in range(NUM_BUCKETS):
        d = deltas[t]
        bit = jnp.int32(1 << t)
        lo, lm = [], []
        for i in range(KNN):
            a_p = pen[i]
            b_p = pen[KNN - 1 - i] + d
            c = a_p <= b_p
            lo.append(jnp.where(c, a_p, b_p))
            lm.append(jnp.where(c, msk[i], msk[KNN - 1 - i] ^ bit))
        for dist in (8, 4, 2, 1):
            nlo, nlm = list(lo), list(lm)
            for i in range(KNN):
                if i & dist:
                    continue
                j = i + dist
                c = lo[i] <= lo[j]
                nlo[i] = jnp.where(c, lo[i], lo[j])
                nlm[i] = jnp.where(c, lm[i], lm[j])
                nlo[j] = jnp.where(c, lo[j], lo[i])
                nlm[j] = jnp.where(c, lm[j], lm[i])
            lo, lm = nlo, nlm
        pen, msk = lo, lm

    es = [jnp.exp(pen[0] - pen[i]) for i in range(KNN)]
    z = es[0]
    for i in range(1, KNN):
        z = z + es[i]
    rz = 1.0 / z
    icat = jnp.concatenate([code ^ msk[i] for i in range(KNN)], axis=0)
    wcat = jnp.concatenate([es[i] * rz for i in range(KNN)], axis=0)
    idx_ref[...] = icat.T
    wts_ref[...] = wcat.T


def _scores_topk(x_flat, Wd, Wu, KK, bd2):
    T = x_flat.shape[0]
    grid = (T // TOK_BLK,)
    return pl.pallas_call(
        _scores_topk_body,
        grid=grid,
        in_specs=[
            pl.BlockSpec((TOK_BLK, D_MODEL), lambda i: (i, 0)),
            pl.BlockSpec((Q_RANK, D_MODEL), lambda i: (0, 0)),
            pl.BlockSpec((HEADS * KEY_DIM, Q_RANK), lambda i: (0, 0)),
            pl.BlockSpec((2 * HM, HEADS * KEY_DIM), lambda i: (0, 0)),
            pl.BlockSpec((Q_RANK, 1), lambda i: (0, 0)),
        ],
        out_specs=[
            pl.BlockSpec((TOK_BLK, HEADS * KNN), lambda i: (i, 0)),
            pl.BlockSpec((TOK_BLK, HEADS * KNN), lambda i: (i, 0)),
        ],
        out_shape=[
            jax.ShapeDtypeStruct((T, HEADS * KNN), jnp.int32),
            jax.ShapeDtypeStruct((T, HEADS * KNN), jnp.float32),
        ],
    )(x_flat, Wd, Wu, KK, bd2)


NW = 32
TW = 128
NB = 4
BAG = HEADS * KNN


def _bag_body(idx_hbm, w_hbm, table_hbm, out_hbm, idx_v, w_v, rows_v, y_v,
              sem0, sem1, sem2, sem3):
    wid = lax.axis_index("s") * 2 + lax.axis_index("c")
    tok0 = wid * TW
    sems = (sem0, sem1, sem2, sem3)

    pltpu.sync_copy(idx_hbm.at[pl.ds(tok0, TW)], idx_v)
    pltpu.sync_copy(w_hbm.at[pl.ds(tok0, TW)], w_v)

    def fire(ci, b, sem):
        pltpu.async_copy(table_hbm.at[idx_v.at[ci]], rows_v.at[b], sem)

    for j in range(NB - 1):
        fire(j, j, sems[j])

    def quad(p, _):
        for b in range(NB):
            ci = p * NB + b

            @pl.when(ci + NB - 1 < TW)
            def _(ci=ci, b=b):
                fire(ci + NB - 1, (b + NB - 1) % NB, sems[(b + NB - 1) % NB])

            pltpu.make_async_copy(
                table_hbm.at[idx_v.at[ci]], rows_v.at[b], sems[b]).wait()

            def grp(g, accs, ci=ci, b=b):
                w16 = w_v[ci, pl.ds(g * 16, 16)]
                for r in range(16):
                    wr = jnp.take(w16, jnp.full((16,), r, jnp.int32))
                    row = g * 16 + r
                    accs = tuple(
                        accs[j] + wr * rows_v[b, row, pl.ds(j * 16, 16)]
                        for j in range(VALUE_DIM // 16)
                    )
                return accs

            accs = lax.fori_loop(
                0, BAG // 16, grp,
                tuple(jnp.zeros((16,), jnp.float32)
                      for _ in range(VALUE_DIM // 16)))
            for j in range(VALUE_DIM // 16):
                y_v[0, pl.ds(j * 16, 16)] = accs[j]
            pltpu.sync_copy(y_v, out_hbm.at[pl.ds(tok0 + ci, 1)])
        return 0

    lax.fori_loop(0, TW // NB, quad, 0)


def _bag(idx_t, w_t, values):
    T = idx_t.shape[0]
    mesh = plsc.VectorSubcoreMesh(core_axis_name="c", subcore_axis_name="s")
    return pl.kernel(
        _bag_body,
        out_type=jax.ShapeDtypeStruct((T, VALUE_DIM), jnp.float32),
        mesh=mesh,
        scratch_types=[
            pltpu.VMEM((TW, BAG), jnp.int32),
            pltpu.VMEM((TW, BAG), jnp.float32),
            pltpu.VMEM((NB, BAG, VALUE_DIM), jnp.float32),
            pltpu.VMEM((1, VALUE_DIM), jnp.float32),
            pltpu.SemaphoreType.DMA,
            pltpu.SemaphoreType.DMA,
            pltpu.SemaphoreType.DMA,
            pltpu.SemaphoreType.DMA,
        ],
    )(idx_t, w_t, values)


def _proj_body(y_ref, wp_ref, out_ref):
    out_ref[...] = jnp.dot(y_ref[...], wp_ref[...],
                           preferred_element_type=jnp.float32)


def _proj(y, WpT):
    T = y.shape[0]
    blk = 512
    return pl.pallas_call(
        _proj_body,
        grid=(T // blk,),
        in_specs=[
            pl.BlockSpec((blk, VALUE_DIM), lambda i: (i, 0)),
            pl.BlockSpec((VALUE_DIM, D_MODEL), lambda i: (0, 0)),
        ],
        out_specs=pl.BlockSpec((blk, D_MODEL), lambda i: (i, 0)),
        out_shape=jax.ShapeDtypeStruct((T, D_MODEL), jnp.float32),
    )(y, WpT)


def kernel(x, keys, values, Wd, bd, Wu, Wp):
    bsz, seq_len, _ = x.shape
    T = bsz * seq_len
    x_flat = x.reshape(T, D_MODEL)

    E = jnp.asarray(_PERM)
    keys_r = keys.transpose(2, 0, 1, 3).reshape(2, HM, BUCKET_DIM)
    KK = (E[None, :, :, None] * keys_r[:, None, :, :]).reshape(2 * HM, HM * BUCKET_DIM)
    bd2 = bd.reshape(Q_RANK, 1)

    idx_t, w_t = _scores_topk(x_flat, Wd, Wu, KK, bd2)

    y = _bag(idx_t, w_t, values)
    out = _proj(y, Wp.T)
    return out.reshape(bsz, seq_len, D_MODEL)
